# baseline (speedup 1.0000x reference)
import functools

import jax
import jax.numpy as jnp
from jax import lax
from jax.experimental import pallas as pl
from jax.experimental.pallas import tpu as pltpu
from jax.experimental.pallas import tpu_sc as plsc

EMBED = 32
LANES = 16
NUM_CORES = 2
NUM_SUBCORES = 16
NUM_WORKERS = NUM_CORES * NUM_SUBCORES


def _mf_body(uid_hbm, mid_hbm, utab_hbm, mtab_hbm, out_hbm,
             idx_u, idx_m, rows_u, rows_m, out_v, sem_u, sem_m):
    b_per_w = idx_u.shape[0]
    wid = lax.axis_index("s") * NUM_CORES + lax.axis_index("c")
    base = wid * b_per_w

    pltpu.sync_copy(uid_hbm.at[pl.ds(base, b_per_w)], idx_u)
    pltpu.sync_copy(mid_hbm.at[pl.ds(base, b_per_w)], idx_m)

    cu = pltpu.async_copy(utab_hbm.at[idx_u], rows_u, sem_u)
    cm = pltpu.async_copy(mtab_hbm.at[idx_m], rows_m, sem_m)
    cu.wait()
    cm.wait()

    jrow = lax.iota(jnp.int32, LANES)

    def chunk_body(c, carry):
        row_idx = jrow + c * LANES
        acc = jnp.zeros((LANES,), jnp.float32)
        for e in range(EMBED):
            col = jnp.full((LANES,), e, jnp.int32)
            uv = plsc.load_gather(rows_u, [row_idx, col])
            mv = plsc.load_gather(rows_m, [row_idx, col])
            acc = acc + uv * mv
        out_v[pl.ds(c * LANES, LANES)] = acc
        return carry

    lax.fori_loop(0, b_per_w // LANES, chunk_body, 0)

    pltpu.sync_copy(out_v, out_hbm.at[pl.ds(base, b_per_w)])


@jax.jit
def kernel(user_ids, movie_ids, user_table, movie_table):
    uid = user_ids.astype(jnp.int32)
    mid = movie_ids.astype(jnp.int32)
    batch = uid.shape[0]
    b_per_w = batch // NUM_WORKERS

    mesh = plsc.VectorSubcoreMesh(
        core_axis_name="c", subcore_axis_name="s",
        num_cores=NUM_CORES, num_subcores=NUM_SUBCORES)

    mf = pl.kernel(
        _mf_body,
        out_type=jax.ShapeDtypeStruct((batch,), jnp.float32),
        mesh=mesh,
        scratch_types=[
            pltpu.VMEM((b_per_w,), jnp.int32),
            pltpu.VMEM((b_per_w,), jnp.int32),
            pltpu.VMEM((b_per_w, EMBED), jnp.float32),
            pltpu.VMEM((b_per_w, EMBED), jnp.float32),
            pltpu.VMEM((b_per_w,), jnp.float32),
            pltpu.SemaphoreType.DMA,
            pltpu.SemaphoreType.DMA,
        ],
        compiler_params=pltpu.CompilerParams(
            needs_layout_passes=False, use_tc_tiling_on_sc=False),
    )
    return mf(uid, mid, user_table, movie_table)
